# baseline (speedup 1.0000x reference)
import functools

import jax
import jax.numpy as jnp
from jax import lax
from jax.experimental import pallas as pl
from jax.experimental.pallas import tpu as pltpu
from jax.experimental.pallas import tpu_sc as plsc

OUT_FEATURES = 4096
IN_FEATURES = 4096
NUM_CORES = 2
NUM_SUBCORES = 16
NW = NUM_CORES * NUM_SUBCORES
ROWS_PER_W = OUT_FEATURES // NW
CROWS = 8
CCOLS = 1024
CHUNK = CROWS * CCOLS
COL_SLABS = IN_FEATURES // CCOLS
NCHUNK = (ROWS_PER_W // CROWS) * COL_SLABS
NC2 = NCHUNK // 2
LANES = 16
UNROLL = 8

_mesh = plsc.VectorSubcoreMesh(core_axis_name="c", subcore_axis_name="s")


@functools.partial(
    pl.kernel,
    mesh=_mesh,
    out_type=jax.ShapeDtypeStruct((OUT_FEATURES, IN_FEATURES), jnp.float32),
    compiler_params=pltpu.CompilerParams(use_tc_tiling_on_sc=True),
    scratch_types=[
        pltpu.VMEM((CROWS, CCOLS), jnp.int32),
        pltpu.VMEM((CROWS, CCOLS), jnp.int32),
        pltpu.VMEM((CHUNK,), jnp.int32),
        pltpu.VMEM((CHUNK,), jnp.int32),
        pltpu.VMEM((CHUNK,), jnp.float32),
        pltpu.VMEM((CHUNK,), jnp.float32),
        pltpu.VMEM((CROWS, CCOLS), jnp.float32),
        pltpu.VMEM((CROWS, CCOLS), jnp.float32),
        pltpu.VMEM((CROWS, CCOLS), jnp.float32),
        pltpu.VMEM((CROWS, CCOLS), jnp.float32),
        pltpu.SemaphoreType.DMA,
        pltpu.SemaphoreType.DMA,
        pltpu.SemaphoreType.DMA,
        pltpu.SemaphoreType.DMA,
        pltpu.SemaphoreType.DMA,
        pltpu.SemaphoreType.DMA,
        pltpu.SemaphoreType.DMA,
        pltpu.SemaphoreType.DMA,
    ],
)
def _ssl_gather(w_hbm, idx_hbm, g_hbm, out_hbm,
                ib0, ib1, il0, il1, wv0, wv1, gb0, gb1, ob0, ob1,
                si0, si1, sw0, sw1, sg0, sg1, so0, so1):
    wid = lax.axis_index("s") * NUM_CORES + lax.axis_index("c")
    base_row = wid * ROWS_PER_W

    def relayout(ib, il):
        for r in range(CROWS):
            def body(j, _):
                for u in range(UNROLL):
                    c = pl.ds(
                        pl.multiple_of((j * UNROLL + u) * LANES, LANES),
                        LANES)
                    p = pl.ds(
                        pl.multiple_of(r * CCOLS + (j * UNROLL + u) * LANES,
                                       LANES), LANES)
                    il[p] = ib[r, c] & 15
                return 0

            lax.fori_loop(0, CCOLS // (LANES * UNROLL), body, 0)

    def multiply(wv, gb, ob):
        for r in range(CROWS):
            def body(j, _):
                for u in range(UNROLL):
                    c = pl.ds(
                        pl.multiple_of((j * UNROLL + u) * LANES, LANES),
                        LANES)
                    p = pl.ds(
                        pl.multiple_of(r * CCOLS + (j * UNROLL + u) * LANES,
                                       LANES), LANES)
                    ob[r, c] = wv[p] * gb[r, c]
                return 0

            lax.fori_loop(0, CCOLS // (LANES * UNROLL), body, 0)

    def chunk_slice(ref, i):
        row = base_row + (i // COL_SLABS) * CROWS
        col = (i % COL_SLABS) * CCOLS
        return ref.at[pl.ds(row, CROWS), pl.ds(col, CCOLS)]

    pltpu.sync_copy(chunk_slice(idx_hbm, 0), ib0)
    relayout(ib0, il0)
    pltpu.async_copy(w_hbm.at[il0], wv0, sw0)
    pltpu.async_copy(chunk_slice(idx_hbm, 1), ib1, si1)
    pltpu.async_copy(chunk_slice(g_hbm, 0), gb0, sg0)
    pltpu.async_copy(chunk_slice(g_hbm, 1), gb1, sg1)

    def half(i, i2, iba, ibb, ila, ilb, wva, wvb, gba, oba,
             sia, sib, swa, swb, sga, soa, first):
        last_pair = i2 == NC2 - 1

        def fire_next():
            pltpu.make_async_copy(chunk_slice(idx_hbm, i + 1), ibb,
                                  sib).wait()
            relayout(ibb, ilb)
            pltpu.async_copy(w_hbm.at[ilb], wvb, swb)

        if first:
            fire_next()
        else:
            pl.when(i2 < NC2 - 1)(fire_next)

        @pl.when(i2 < NC2 - 1)
        def _():
            pltpu.async_copy(chunk_slice(idx_hbm, i + 2), iba, sia)

        pltpu.make_async_copy(w_hbm.at[ila], wva, swa).wait()
        pltpu.make_async_copy(chunk_slice(g_hbm, i), gba, sga).wait()
        if first:
            @pl.when(i2 > 0)
            def _():
                pltpu.make_async_copy(oba, chunk_slice(out_hbm, i - 2),
                                      soa).wait()
        else:
            @pl.when(i2 > 0)
            def _():
                pltpu.make_async_copy(oba, chunk_slice(out_hbm, i - 2),
                                      soa).wait()

        multiply(wva, gba, oba)
        pltpu.async_copy(oba, chunk_slice(out_hbm, i), soa)

        @pl.when(i2 < NC2 - 1)
        def _():
            pltpu.async_copy(chunk_slice(g_hbm, i + 2), gba, sga)

    def pair_body(i2, _):
        i = i2 * 2
        half(i, i2, ib0, ib1, il0, il1, wv0, wv1, gb0, ob0,
             si0, si1, sw0, sw1, sg0, so0, first=True)
        half(i + 1, i2, ib1, ib0, il1, il0, wv1, wv0, gb1, ob1,
             si1, si0, sw1, sw0, sg1, so1, first=False)
        return 0

    lax.fori_loop(0, NC2, pair_body, 0)

    pltpu.make_async_copy(ob0, chunk_slice(out_hbm, NCHUNK - 2), so0).wait()
    pltpu.make_async_copy(ob1, chunk_slice(out_hbm, NCHUNK - 1), so1).wait()


def kernel(weight, IDX, G):
    return _ssl_gather(weight, IDX, G)


# speedup vs baseline: 49.3403x; 49.3403x over previous
import functools

import jax
import jax.numpy as jnp
from jax import lax
from jax.experimental import pallas as pl
from jax.experimental.pallas import tpu as pltpu
from jax.experimental.pallas import tpu_sc as plsc

OUT_FEATURES = 4096
IN_FEATURES = 4096
NUM_CORES = 2
NUM_SUBCORES = 16
NW = NUM_CORES * NUM_SUBCORES
ROWS_PER_W = OUT_FEATURES // NW
CROWS = 8
CCOLS = 256
CHUNK = CROWS * CCOLS
COL_SLABS = IN_FEATURES // CCOLS
NCHUNK = (ROWS_PER_W // CROWS) * COL_SLABS
NC2 = NCHUNK // 2
LANES = 16
VECS = CCOLS // LANES
GRAIN = 512
NSUB = CHUNK // GRAIN
SLICE = 1441792
FLAG = 1 << 30

_mesh = plsc.VectorSubcoreMesh(core_axis_name="c", subcore_axis_name="s")


@functools.partial(
    pl.kernel,
    mesh=_mesh,
    out_type=jax.ShapeDtypeStruct((OUT_FEATURES, IN_FEATURES), jnp.float32),
    compiler_params=pltpu.CompilerParams(use_tc_tiling_on_sc=True,
                                         needs_layout_passes=False),
    scratch_types=[
        pltpu.VMEM((CROWS, CCOLS), jnp.int32),
        pltpu.VMEM((CROWS, CCOLS), jnp.int32),
        pltpu.VMEM((CHUNK,), jnp.int32),
        pltpu.VMEM((CHUNK,), jnp.int32),
        pltpu.VMEM((CHUNK,), jnp.int32),
        pltpu.VMEM((CHUNK,), jnp.int32),
        pltpu.VMEM((CHUNK,), jnp.int32),
        pltpu.VMEM((CHUNK,), jnp.int32),
        pltpu.VMEM((CHUNK,), jnp.float32),
        pltpu.VMEM((CHUNK,), jnp.float32),
        pltpu.VMEM((CHUNK,), jnp.float32),
        pltpu.VMEM((CHUNK,), jnp.float32),
        pltpu.VMEM((CROWS, CCOLS), jnp.float32),
        pltpu.VMEM((CROWS, CCOLS), jnp.float32),
        pltpu.VMEM((CROWS, CCOLS), jnp.float32),
        pltpu.VMEM((CROWS, CCOLS), jnp.float32),
        pltpu.VMEM_SHARED((SLICE,), jnp.float32),
        pltpu.SemaphoreType.DMA,
        pltpu.SemaphoreType.DMA,
        pltpu.SemaphoreType.DMA,
        pltpu.SemaphoreType.DMA,
        pltpu.SemaphoreType.DMA,
        pltpu.SemaphoreType.DMA,
        pltpu.SemaphoreType.DMA,
        pltpu.SemaphoreType.DMA,
        pltpu.SemaphoreType.DMA,
        pltpu.SemaphoreType.DMA,
    ],
)
def _ssl_gather(w_hbm, idx_hbm, g_hbm, out_hbm,
                ib0, ib1, sl0, sl1, la0, la1, lb0, lb1,
                wa0, wa1, wb0, wb1, gb0, gb1, ob0, ob1, w_sh,
                si0, si1, sa0, sa1, sb0, sb1, sg0, sg1, so0, so1):
    wid = lax.axis_index("s") * NUM_CORES + lax.axis_index("c")
    sid = lax.axis_index("s")
    base_row = wid * ROWS_PER_W
    iota = lax.iota(jnp.int32, LANES)

    def init_list(lst):
        def body(j, _):
            lst[pl.ds(pl.multiple_of(j * LANES, LANES), LANES)] = (
                (iota + j * LANES) * 128) & 0xFFFFF
            return 0

        lax.fori_loop(0, CHUNK // LANES, body, 0)

    for lst in (la0, la1, lb0, lb1):
        init_list(lst)

    part = SLICE // NUM_SUBCORES
    off = sid * part
    pltpu.sync_copy(w_hbm.at[pl.ds(off, part)], w_sh.at[pl.ds(off, part)])
    plsc.subcore_barrier()

    def chunk_slice(ref, i):
        row = base_row + (i // COL_SLABS) * CROWS
        col = (i % COL_SLABS) * CCOLS
        return ref.at[pl.ds(row, CROWS), pl.ds(col, CCOLS)]

    def partition(ib, sl, la, lb):
        zero = jnp.zeros((LANES,), jnp.int32)
        carry = (zero, zero)
        for r in range(CROWS):
            def body(jc, cr):
                base_a, base_b = cr
                c = pl.ds(pl.multiple_of(jc * LANES, LANES), LANES)
                p = pl.ds(
                    pl.multiple_of(r * CCOLS + jc * LANES, LANES), LANES)
                v = ib[r, c]
                m = v < SLICE
                mi = jnp.where(m, 1, 0).astype(jnp.int32)
                cs = plsc.cumsum(mi)
                slot_a = jnp.maximum(base_a + cs - 1, 0)
                slot_b = jnp.maximum(base_b + iota - cs, 0)
                sl[p] = jnp.where(m, slot_a + FLAG, slot_b)
                plsc.store_scatter(la, [slot_a], v, mask=m)
                plsc.store_scatter(lb, [slot_b], v, mask=jnp.logical_not(m))
                pop = plsc.all_reduce_population_count(m)
                return (base_a + pop, base_b + (16 - pop))

            carry = lax.fori_loop(0, VECS, body, carry)
        return lax.reduce_max(carry[0], (0,))

    def fire_gathers(la, lb, wa, wb, sa, sb, n_a):
        pltpu.async_copy(w_sh.at[la], wa, sa)
        n_b = CHUNK - n_a
        for k in range(NSUB):
            s = pl.ds(k * GRAIN, GRAIN)

            @pl.when(n_b > k * GRAIN)
            def _():
                pltpu.async_copy(w_hbm.at[lb.at[s]], wb.at[s], sb)

    def wait_gathers(wa, sa, sb, n_a):
        pltpu.make_async_copy(w_hbm.at[pl.ds(0, CHUNK)], wa, sa).wait()
        n_b = CHUNK - n_a
        s = pl.ds(0, GRAIN)
        for k in range(NSUB):
            @pl.when(n_b > k * GRAIN)
            def _():
                pltpu.make_async_copy(w_hbm.at[s], wa.at[s], sb).wait()

    def merge(sl, wa, wb, gb, ob):
        for r in range(CROWS):
            def body(jc, _):
                c = pl.ds(pl.multiple_of(jc * LANES, LANES), LANES)
                p = pl.ds(
                    pl.multiple_of(r * CCOLS + jc * LANES, LANES), LANES)
                enc = sl[p]
                m = enc >= FLAG
                slot = jnp.where(m, enc - FLAG, enc)
                va = plsc.load_gather(wa, [slot])
                vb = plsc.load_gather(wb, [slot])
                ob[r, c] = jnp.where(m, va, vb) * gb[r, c]
                return 0

            lax.fori_loop(0, VECS, body, 0)

    bufs = ((ib0, sl0, la0, lb0, wa0, wb0, gb0, ob0, si0, sa0, sb0, sg0, so0),
            (ib1, sl1, la1, lb1, wa1, wb1, gb1, ob1, si1, sa1, sb1, sg1, so1))

    def half(i, par, n_as, fire_next=True, prefetch=True, wait_out=True):
        a = bufs[par]
        b = bufs[1 - par]
        iba, sla, laa, lba, waa, wba, gba, oba, sia, saa, sba, sga, soa = a
        ibb, slb, lab, lbb, wab, wbb, gbb, obb, sib, sab, sbb, sgb, sob = b
        n_a_cur = n_as[par]
        n_a_next = n_as[1 - par]

        if fire_next:
            pltpu.make_async_copy(chunk_slice(idx_hbm, i + 1), ibb,
                                  sib).wait()
            n_a_next = partition(ibb, slb, lab, lbb)
            fire_gathers(lab, lbb, wab, wbb, sab, sbb, n_a_next)

        if prefetch:
            pltpu.async_copy(chunk_slice(idx_hbm, i + 2), iba, sia)

        wait_gathers(waa, saa, sba, n_a_cur)
        pltpu.make_async_copy(chunk_slice(g_hbm, i), gba, sga).wait()
        if wait_out:
            pltpu.make_async_copy(oba, chunk_slice(out_hbm, i - 2),
                                  soa).wait()

        merge(sla, waa, wba, gba, oba)
        pltpu.async_copy(oba, chunk_slice(out_hbm, i), soa)

        if prefetch:
            pltpu.async_copy(chunk_slice(g_hbm, i + 2), gba, sga)

        if par == 0:
            return (n_a_cur, n_a_next)
        return (n_a_next, n_a_cur)

    pltpu.sync_copy(chunk_slice(idx_hbm, 0), ib0)
    n_a0 = partition(ib0, sl0, la0, lb0)
    fire_gathers(la0, lb0, wa0, wb0, sa0, sb0, n_a0)
    pltpu.async_copy(chunk_slice(idx_hbm, 1), ib1, si1)
    pltpu.async_copy(chunk_slice(g_hbm, 0), gb0, sg0)
    pltpu.async_copy(chunk_slice(g_hbm, 1), gb1, sg1)

    n_as = (n_a0, jnp.int32(0))
    n_as = half(0, 0, n_as, wait_out=False)
    n_as = half(1, 1, n_as, wait_out=False)

    def pair_body(i2, n_as):
        i = i2 * 2
        n_as = half(i, 0, n_as)
        n_as = half(i + 1, 1, n_as)
        return n_as

    n_as = lax.fori_loop(1, NC2 - 1, pair_body, n_as)

    n_as = half(NCHUNK - 2, 0, n_as, prefetch=False)
    n_as = half(NCHUNK - 1, 1, n_as, fire_next=False, prefetch=False)

    pltpu.make_async_copy(ob0, chunk_slice(out_hbm, NCHUNK - 2), so0).wait()
    pltpu.make_async_copy(ob1, chunk_slice(out_hbm, NCHUNK - 1), so1).wait()


def kernel(weight, IDX, G):
    return _ssl_gather(weight, IDX, G)


# speedup vs baseline: 116.7508x; 2.3662x over previous
import functools

import jax
import jax.numpy as jnp
from jax import lax
from jax.experimental import pallas as pl
from jax.experimental.pallas import tpu as pltpu
from jax.experimental.pallas import tpu_sc as plsc

OUT_FEATURES = 4096
IN_FEATURES = 4096
NUM_CORES = 2
NUM_SUBCORES = 16
NW = NUM_CORES * NUM_SUBCORES
ROWS_PER_W = OUT_FEATURES // NW
CROWS = 8
CCOLS = 1024
CHUNK = CROWS * CCOLS
COL_SLABS = IN_FEATURES // CCOLS
NCHUNK = (ROWS_PER_W // CROWS) * COL_SLABS
NC2 = NCHUNK // 2
LANES = 16
UNROLL = 8

_mesh = plsc.VectorSubcoreMesh(core_axis_name="c", subcore_axis_name="s")


@functools.partial(
    pl.kernel,
    mesh=_mesh,
    out_type=jax.ShapeDtypeStruct((OUT_FEATURES, IN_FEATURES), jnp.float32),
    compiler_params=pltpu.CompilerParams(use_tc_tiling_on_sc=True),
    scratch_types=[
        pltpu.VMEM((CROWS, CCOLS), jnp.int32),
        pltpu.VMEM((CROWS, CCOLS), jnp.int32),
        pltpu.VMEM((CHUNK,), jnp.int32),
        pltpu.VMEM((CHUNK,), jnp.int32),
        pltpu.VMEM((CHUNK,), jnp.float32),
        pltpu.VMEM((CHUNK,), jnp.float32),
        pltpu.VMEM((CROWS, CCOLS), jnp.float32),
        pltpu.VMEM((CROWS, CCOLS), jnp.float32),
        pltpu.VMEM((CROWS, CCOLS), jnp.float32),
        pltpu.VMEM((CROWS, CCOLS), jnp.float32),
        pltpu.SemaphoreType.DMA,
        pltpu.SemaphoreType.DMA,
        pltpu.SemaphoreType.DMA,
        pltpu.SemaphoreType.DMA,
        pltpu.SemaphoreType.DMA,
        pltpu.SemaphoreType.DMA,
        pltpu.SemaphoreType.DMA,
        pltpu.SemaphoreType.DMA,
    ],
)
def _ssl_gather(w_hbm, idx_hbm, g_hbm, out_hbm,
                ib0, ib1, il0, il1, wv0, wv1, gb0, gb1, ob0, ob1,
                si0, si1, sw0, sw1, sg0, sg1, so0, so1):
    wid = lax.axis_index("s") * NUM_CORES + lax.axis_index("c")
    base_row = wid * ROWS_PER_W

    def relayout(ib, il):
        for r in range(CROWS):
            def body(j, _):
                for u in range(UNROLL):
                    c = pl.ds(
                        pl.multiple_of((j * UNROLL + u) * LANES, LANES),
                        LANES)
                    p = pl.ds(
                        pl.multiple_of(r * CCOLS + (j * UNROLL + u) * LANES,
                                       LANES), LANES)
                    il[p] = ib[r, c]
                return 0

            lax.fori_loop(0, CCOLS // (LANES * UNROLL), body, 0)

    def multiply(wv, gb, ob):
        for r in range(CROWS):
            def body(j, _):
                for u in range(UNROLL):
                    c = pl.ds(
                        pl.multiple_of((j * UNROLL + u) * LANES, LANES),
                        LANES)
                    p = pl.ds(
                        pl.multiple_of(r * CCOLS + (j * UNROLL + u) * LANES,
                                       LANES), LANES)
                    ob[r, c] = wv[p] * gb[r, c]
                return 0

            lax.fori_loop(0, CCOLS // (LANES * UNROLL), body, 0)

    def chunk_slice(ref, i):
        row = base_row + (i // COL_SLABS) * CROWS
        col = (i % COL_SLABS) * CCOLS
        return ref.at[pl.ds(row, CROWS), pl.ds(col, CCOLS)]

    pltpu.sync_copy(chunk_slice(idx_hbm, 0), ib0)
    relayout(ib0, il0)
    pltpu.async_copy(w_hbm.at[il0], wv0, sw0)
    pltpu.async_copy(chunk_slice(idx_hbm, 1), ib1, si1)
    pltpu.async_copy(chunk_slice(g_hbm, 0), gb0, sg0)
    pltpu.async_copy(chunk_slice(g_hbm, 1), gb1, sg1)

    def half(i, i2, iba, ibb, ila, ilb, wva, wvb, gba, oba,
             sia, sib, swa, swb, sga, soa, first):
        last_pair = i2 == NC2 - 1

        def fire_next():
            pltpu.make_async_copy(chunk_slice(idx_hbm, i + 1), ibb,
                                  sib).wait()
            relayout(ibb, ilb)
            pltpu.async_copy(w_hbm.at[ilb], wvb, swb)

        if first:
            fire_next()
        else:
            pl.when(i2 < NC2 - 1)(fire_next)

        @pl.when(i2 < NC2 - 1)
        def _():
            pltpu.async_copy(chunk_slice(idx_hbm, i + 2), iba, sia)

        pltpu.make_async_copy(w_hbm.at[ila], wva, swa).wait()
        pltpu.make_async_copy(chunk_slice(g_hbm, i), gba, sga).wait()
        if first:
            @pl.when(i2 > 0)
            def _():
                pltpu.make_async_copy(oba, chunk_slice(out_hbm, i - 2),
                                      soa).wait()
        else:
            @pl.when(i2 > 0)
            def _():
                pltpu.make_async_copy(oba, chunk_slice(out_hbm, i - 2),
                                      soa).wait()

        multiply(wva, gba, oba)
        pltpu.async_copy(oba, chunk_slice(out_hbm, i), soa)

        @pl.when(i2 < NC2 - 1)
        def _():
            pltpu.async_copy(chunk_slice(g_hbm, i + 2), gba, sga)

    def pair_body(i2, _):
        i = i2 * 2
        half(i, i2, ib0, ib1, il0, il1, wv0, wv1, gb0, ob0,
             si0, si1, sw0, sw1, sg0, so0, first=True)
        half(i + 1, i2, ib1, ib0, il1, il0, wv1, wv0, gb1, ob1,
             si1, si0, sw1, sw0, sg1, so1, first=False)
        return 0

    lax.fori_loop(0, NC2, pair_body, 0)

    pltpu.make_async_copy(ob0, chunk_slice(out_hbm, NCHUNK - 2), so0).wait()
    pltpu.make_async_copy(ob1, chunk_slice(out_hbm, NCHUNK - 1), so1).wait()


def kernel(weight, IDX, G):
    return _ssl_gather(weight, IDX, G)
